# baseline (speedup 1.0000x reference)
import functools

import jax
import jax.numpy as jnp
from jax import lax
from jax.experimental import pallas as pl
from jax.experimental.pallas import tpu as pltpu
from jax.experimental.pallas import tpu_sc as plsc

_T = 5
_NB = 4

_GDN = lax.GatherDimensionNumbers(
    offset_dims=(), collapsed_slice_dims=(0,), start_index_map=(0,))


def _sc_body(idx_hbm, tok_hbm, pos_hbm, out_hbm,
             tok_v, pos_v, iv, bbv, ib0, ib1, roe, roo,
             semi0, semi1, semoe, semoo):
    D = tok_hbm.shape[1]
    L = pos_hbm.shape[0]
    B = idx_hbm.shape[0] // L
    c = lax.axis_index("c")
    s = lax.axis_index("s")
    LW = L // 32
    NG = (_NB * LW) // 16
    wid = c * 16 + s
    l0 = wid * LW

    pltpu.sync_copy(tok_hbm, tok_v)
    pltpu.sync_copy(pos_hbm.at[pl.ds(l0, LW), :], pos_v)

    NC = D // 16
    tok_c = [[tok_v[t, pl.ds(dc * 16, 16)] for dc in range(NC)]
             for t in range(_T)]

    def idx_issue(b0, ib, sem):
        for r in range(_NB):
            pltpu.async_copy(idx_hbm.at[pl.ds((b0 + r) * L + l0, LW)],
                             ib.at[pl.ds(r * LW, LW)], sem)

    def idx_wait(ib, sem):
        for r in range(_NB):
            pltpu.make_async_copy(idx_hbm.at[pl.ds(l0, LW)],
                                  ib.at[pl.ds(0, LW)], sem).wait()

    def out_issue(b0, ro, sem):
        pltpu.async_copy(ro, out_hbm.at[pl.ds(b0, _NB)].at[:, pl.ds(l0, LW), :],
                         sem)

    def out_wait(ro, sem):
        pltpu.make_async_copy(ro, out_hbm.at[pl.ds(0, _NB)].at[:, pl.ds(l0, LW), :],
                              sem).wait()

    def fill(ro):
        def group(g, carry):
            idxg = iv[pl.ds(g * 16, 16)]
            h = g // (LW // 16)
            jb = (g % (LW // 16)) * 16
            for u in range(16):
                jrow = jb + u
                bb = lax.gather(idxg, jnp.full((16, 1), u, jnp.int32), _GDN,
                                (1,), mode=lax.GatherScatterMode.PROMISE_IN_BOUNDS)
                bbv[pl.ds(0, 16)] = bb
                bbl = bbv[pl.ds(0, 16)]
                ms = [bbl == t for t in range(1, _T)]
                for dc in range(NC):
                    sl = pl.ds(dc * 16, 16)
                    val = tok_c[0][dc]
                    for t in range(1, _T):
                        val = jnp.where(ms[t - 1], tok_c[t][dc], val)
                    ro[h, jrow, sl] = val + pos_v[jrow, sl]
            return carry

        lax.fori_loop(0, NG, group, 0)

    def stage(ib):
        for g in range(NG):
            sl = pl.ds(g * 16, 16)
            iv[sl] = ib[sl]

    idx_issue(0, ib0, semi0)
    idx_issue(_NB, ib1, semi1)

    KL = B // (2 * _NB)

    def loop(k, carry):
        b0 = k * 2 * _NB

        idx_wait(ib0, semi0)
        stage(ib0)

        @pl.when(k < KL - 1)
        def _w1():
            idx_issue(b0 + 2 * _NB, ib0, semi0)

        @pl.when(k > 0)
        def _w0():
            out_wait(roe, semoe)

        fill(roe)
        out_issue(b0, roe, semoe)

        idx_wait(ib1, semi1)
        stage(ib1)

        @pl.when(k < KL - 1)
        def _w3():
            idx_issue(b0 + 3 * _NB, ib1, semi1)

        @pl.when(k > 0)
        def _w2():
            out_wait(roo, semoo)

        fill(roo)
        out_issue(b0 + _NB, roo, semoo)
        return carry

    lax.fori_loop(0, KL, loop, 0)
    out_wait(roe, semoe)
    out_wait(roo, semoo)


def kernel(seq_indices, token_embed, pos_embed):
    B, L = seq_indices.shape
    D = token_embed.shape[1]
    LW = L // 32
    k = functools.partial(
        pl.kernel,
        out_type=jax.ShapeDtypeStruct((B, L, D), jnp.float32),
        mesh=plsc.VectorSubcoreMesh(core_axis_name="c", subcore_axis_name="s"),
        scratch_types=[
            pltpu.VMEM((_T, D), jnp.float32),
            pltpu.VMEM((LW, D), jnp.float32),
            pltpu.VMEM((_NB * LW,), jnp.int32),
            pltpu.VMEM((16,), jnp.int32),
            pltpu.VMEM((_NB * LW,), jnp.int32),
            pltpu.VMEM((_NB * LW,), jnp.int32),
            pltpu.VMEM((_NB, LW, D), jnp.float32),
            pltpu.VMEM((_NB, LW, D), jnp.float32),
            pltpu.SemaphoreType.DMA,
            pltpu.SemaphoreType.DMA,
            pltpu.SemaphoreType.DMA,
            pltpu.SemaphoreType.DMA,
        ],
    )(_sc_body)
    return k(seq_indices.reshape(-1), token_embed, pos_embed[:L])


# speedup vs baseline: 1.6053x; 1.6053x over previous
import functools

import jax
import jax.numpy as jnp
from jax import lax
from jax.experimental import pallas as pl
from jax.experimental.pallas import tpu as pltpu
from jax.experimental.pallas import tpu_sc as plsc

_T = 5

_GDN = lax.GatherDimensionNumbers(
    offset_dims=(), collapsed_slice_dims=(0,), start_index_map=(0,))


def _sc_body(idx_hbm, tok_hbm, pos_hbm, out_hbm,
             tok_v, pos_v, iv, bbv, ib0, ib1, roe, roo,
             semi0, semi1, semoe, semoo):
    D = tok_hbm.shape[1]
    L = pos_hbm.shape[0]
    B = idx_hbm.shape[0] // L
    c = lax.axis_index("c")
    s = lax.axis_index("s")
    LW = L // 32
    NG = LW // 16
    wid = c * 16 + s
    l0 = wid * LW

    pltpu.sync_copy(tok_hbm, tok_v)
    pltpu.sync_copy(pos_hbm.at[pl.ds(l0, LW), :], pos_v)

    NC = D // 16
    tok_c = [[tok_v[t, pl.ds(dc * 16, 16)] for dc in range(NC)]
             for t in range(_T)]

    def idx_issue(b, ib, sem):
        pltpu.async_copy(idx_hbm.at[pl.ds(b * L + l0, LW)], ib, sem)

    def idx_wait(ib, sem):
        pltpu.make_async_copy(idx_hbm.at[pl.ds(l0, LW)], ib, sem).wait()

    def out_issue(b, ro, sem):
        pltpu.async_copy(ro, out_hbm.at[b].at[pl.ds(l0, LW), :], sem)

    def out_wait(ro, sem):
        pltpu.make_async_copy(ro, out_hbm.at[0].at[pl.ds(l0, LW), :], sem).wait()

    def fill(ro):
        def group(g, carry):
            idxg = iv[pl.ds(g * 16, 16)]
            for u in range(16):
                jrow = g * 16 + u
                bb = lax.gather(idxg, jnp.full((16, 1), u, jnp.int32), _GDN,
                                (1,), mode=lax.GatherScatterMode.PROMISE_IN_BOUNDS)
                bbv[pl.ds(0, 16)] = bb
                bbl = bbv[pl.ds(0, 16)]
                ms = [bbl == t for t in range(1, _T)]
                for dc in range(NC):
                    sl = pl.ds(dc * 16, 16)
                    val = tok_c[0][dc]
                    for t in range(1, _T):
                        val = jnp.where(ms[t - 1], tok_c[t][dc], val)
                    ro[jrow, sl] = val + pos_v[jrow, sl]
            return carry

        lax.fori_loop(0, NG, group, 0)

    def stage(ib, sem):
        for g in range(NG):
            sl = pl.ds(g * 16, 16)
            iv[sl] = ib[sl]

    idx_issue(0, ib0, semi0)
    idx_issue(1, ib1, semi1)

    KL = B // 2

    def loop(k, carry):
        b0 = k * 2

        idx_wait(ib0, semi0)
        stage(ib0, semi0)

        @pl.when(k < KL - 1)
        def _w1():
            idx_issue(b0 + 2, ib0, semi0)

        @pl.when(k > 0)
        def _w0():
            out_wait(roe, semoe)

        fill(roe)
        out_issue(b0, roe, semoe)

        idx_wait(ib1, semi1)
        stage(ib1, semi1)

        @pl.when(k < KL - 1)
        def _w3():
            idx_issue(b0 + 3, ib1, semi1)

        @pl.when(k > 0)
        def _w2():
            out_wait(roo, semoo)

        fill(roo)
        out_issue(b0 + 1, roo, semoo)
        return carry

    lax.fori_loop(0, KL, loop, 0)
    out_wait(roe, semoe)
    out_wait(roo, semoo)


def kernel(seq_indices, token_embed, pos_embed):
    B, L = seq_indices.shape
    D = token_embed.shape[1]
    LW = L // 32
    k = functools.partial(
        pl.kernel,
        out_type=jax.ShapeDtypeStruct((B, L, D), jnp.float32),
        mesh=plsc.VectorSubcoreMesh(core_axis_name="c", subcore_axis_name="s"),
        scratch_types=[
            pltpu.VMEM((_T, D), jnp.float32),
            pltpu.VMEM((LW, D), jnp.float32),
            pltpu.VMEM((LW,), jnp.int32),
            pltpu.VMEM((16,), jnp.int32),
            pltpu.VMEM((LW,), jnp.int32),
            pltpu.VMEM((LW,), jnp.int32),
            pltpu.VMEM((LW, D), jnp.float32),
            pltpu.VMEM((LW, D), jnp.float32),
            pltpu.SemaphoreType.DMA,
            pltpu.SemaphoreType.DMA,
            pltpu.SemaphoreType.DMA,
            pltpu.SemaphoreType.DMA,
        ],
    )(_sc_body)
    return k(seq_indices.reshape(-1), token_embed, pos_embed[:L])
